# baseline (speedup 1.0000x reference)
import functools

import jax
import jax.numpy as jnp
from jax.experimental import pallas as pl

TOP_K = 8
BT = 512


def _router_block(x_ref, w1_ref, b1_ref, w2_ref, b2_ref, w3_ref, b3_ref,
                  idx_ref, wgt_ref):
    x = x_ref[...]
    h = jnp.dot(x, w1_ref[...], preferred_element_type=jnp.float32)
    h = jnp.maximum(h + b1_ref[...], 0.0)
    h = jnp.dot(h, w2_ref[...], preferred_element_type=jnp.float32)
    h = jnp.maximum(h + b2_ref[...], 0.0)
    s = jnp.dot(h, w3_ref[...], preferred_element_type=jnp.float32)
    s = s + b3_ref[...]

    num_e = s.shape[-1]
    lane = jax.lax.broadcasted_iota(jnp.int32, s.shape, 1)
    vals = []
    idxs = []
    for _ in range(TOP_K):
        mx = jnp.max(s, axis=1, keepdims=True)
        im = jnp.min(jnp.where(s == mx, lane, num_e), axis=1, keepdims=True)
        vals.append(mx)
        idxs.append(im)
        s = jnp.where(lane == im, -jnp.inf, s)
    v = jnp.concatenate(vals, axis=1)
    i = jnp.concatenate(idxs, axis=1)
    e = jnp.exp(v - v[:, :1])
    w = e / jnp.sum(e, axis=1, keepdims=True)
    idx_ref[...] = i
    wgt_ref[...] = w


@jax.jit
def _run(x, w1, b1, w2, b2, w3, b3):
    b, d = x.shape
    h = w1.shape[1]
    e = w3.shape[1]
    grid = (b // BT,)
    return pl.pallas_call(
        _router_block,
        grid=grid,
        in_specs=[
            pl.BlockSpec((BT, d), lambda i: (i, 0)),
            pl.BlockSpec((d, h), lambda i: (0, 0)),
            pl.BlockSpec((1, h), lambda i: (0, 0)),
            pl.BlockSpec((h, h), lambda i: (0, 0)),
            pl.BlockSpec((1, h), lambda i: (0, 0)),
            pl.BlockSpec((h, e), lambda i: (0, 0)),
            pl.BlockSpec((1, e), lambda i: (0, 0)),
        ],
        out_specs=[
            pl.BlockSpec((BT, TOP_K), lambda i: (i, 0)),
            pl.BlockSpec((BT, TOP_K), lambda i: (i, 0)),
        ],
        out_shape=[
            jax.ShapeDtypeStruct((b, TOP_K), jnp.int32),
            jax.ShapeDtypeStruct((b, TOP_K), jnp.float32),
        ],
    )(x, w1, b1, w2, b2, w3, b3)


def kernel(prompt_embedding, W1, b1, W2, b2, W3, b3):
    idx, wgt = _run(prompt_embedding.astype(jnp.float32), W1,
                    b1.reshape(1, -1), W2, b2.reshape(1, -1), W3,
                    b3.reshape(1, -1))
    return idx, wgt, idx[:, 0]


# speedup vs baseline: 1.2145x; 1.2145x over previous
import functools

import jax
import jax.numpy as jnp
from jax.experimental import pallas as pl

TOP_K = 8
BT = 512


def _router_block(x_ref, w1_ref, b1_ref, w2_ref, b2_ref, w3_ref, b3_ref,
                  idx_ref, wgt_ref):
    x = x_ref[...]
    h = jnp.dot(x, w1_ref[...], preferred_element_type=jnp.float32)
    h = jnp.maximum(h + b1_ref[...], 0.0)
    h = jnp.dot(h, w2_ref[...], preferred_element_type=jnp.float32)
    h = jnp.maximum(h + b2_ref[...], 0.0)
    s = jnp.dot(h, w3_ref[...], preferred_element_type=jnp.float32)
    s = s + b3_ref[...]

    lane = jax.lax.broadcasted_iota(jnp.int32, s.shape, 1)
    bits = jax.lax.bitcast_convert_type(s, jnp.int32)
    key = bits ^ (jnp.right_shift(bits, 31) & jnp.int32(0x7FFFFFFF))
    key = (key & jnp.int32(~63)) | (jnp.int32(63) - lane)

    kmaxs = []
    neg_inf_key = jnp.int32(-2**31)
    for _ in range(TOP_K):
        kmax = jnp.max(key, axis=1, keepdims=True)
        kmaxs.append(kmax)
        key = jnp.where(key == kmax, neg_inf_key, key)
    kk = jnp.concatenate(kmaxs, axis=1)

    i = jnp.int32(63) - (kk & jnp.int32(63))
    vbits = kk & jnp.int32(~63)
    vbits = vbits ^ (jnp.right_shift(vbits, 31) & jnp.int32(0x7FFFFFFF))
    v = jax.lax.bitcast_convert_type(vbits, jnp.float32)
    e = jnp.exp(v - v[:, :1])
    w = e / jnp.sum(e, axis=1, keepdims=True)
    idx_ref[...] = i
    wgt_ref[...] = w


@jax.jit
def _run(x, w1, b1, w2, b2, w3, b3):
    b, d = x.shape
    h = w1.shape[1]
    e = w3.shape[1]
    grid = (b // BT,)
    return pl.pallas_call(
        _router_block,
        grid=grid,
        in_specs=[
            pl.BlockSpec((BT, d), lambda i: (i, 0)),
            pl.BlockSpec((d, h), lambda i: (0, 0)),
            pl.BlockSpec((1, h), lambda i: (0, 0)),
            pl.BlockSpec((h, h), lambda i: (0, 0)),
            pl.BlockSpec((1, h), lambda i: (0, 0)),
            pl.BlockSpec((h, e), lambda i: (0, 0)),
            pl.BlockSpec((1, e), lambda i: (0, 0)),
        ],
        out_specs=[
            pl.BlockSpec((BT, TOP_K), lambda i: (i, 0)),
            pl.BlockSpec((BT, TOP_K), lambda i: (i, 0)),
        ],
        out_shape=[
            jax.ShapeDtypeStruct((b, TOP_K), jnp.int32),
            jax.ShapeDtypeStruct((b, TOP_K), jnp.float32),
        ],
    )(x, w1, b1, w2, b2, w3, b3)


def kernel(prompt_embedding, W1, b1, W2, b2, W3, b3):
    idx, wgt = _run(prompt_embedding.astype(jnp.float32), W1,
                    b1.reshape(1, -1), W2, b2.reshape(1, -1), W3,
                    b3.reshape(1, -1))
    return idx, wgt, idx[:, 0]


# speedup vs baseline: 1.2178x; 1.0028x over previous
import functools

import jax
import jax.numpy as jnp
from jax.experimental import pallas as pl

TOP_K = 8
BT = 512


def _router_block(x_ref, w1_ref, b1_ref, w2_ref, b2_ref, w3_ref, b3_ref,
                  idx_ref, wgt_ref):
    x = x_ref[...]
    h = jnp.dot(x, w1_ref[...], preferred_element_type=jnp.float32)
    h = jnp.maximum(h + b1_ref[...], 0.0)
    h = jnp.dot(h, w2_ref[...], preferred_element_type=jnp.float32)
    h = jnp.maximum(h + b2_ref[...], 0.0)
    s = jnp.dot(h, w3_ref[...], preferred_element_type=jnp.float32)
    s = s + b3_ref[...]

    num_e = s.shape[-1]
    flane = jax.lax.broadcasted_iota(jnp.int32, s.shape, 1).astype(jnp.float32)
    vals = []
    idxs = []
    for _ in range(TOP_K):
        mx = jnp.max(s, axis=1, keepdims=True)
        imf = jnp.min(jnp.where(s == mx, flane, jnp.float32(num_e)),
                      axis=1, keepdims=True)
        vals.append(mx)
        idxs.append(imf)
        s = jnp.where(flane == imf, -jnp.inf, s)
    v = jnp.concatenate(vals, axis=1)
    i = jnp.concatenate(idxs, axis=1).astype(jnp.int32)
    e = jnp.exp(v - v[:, :1])
    w = e / jnp.sum(e, axis=1, keepdims=True)
    idx_ref[...] = i
    wgt_ref[...] = w


@jax.jit
def _run(x, w1, b1, w2, b2, w3, b3):
    b, d = x.shape
    h = w1.shape[1]
    e = w3.shape[1]
    grid = (b // BT,)
    return pl.pallas_call(
        _router_block,
        grid=grid,
        in_specs=[
            pl.BlockSpec((BT, d), lambda i: (i, 0)),
            pl.BlockSpec((d, h), lambda i: (0, 0)),
            pl.BlockSpec((1, h), lambda i: (0, 0)),
            pl.BlockSpec((h, h), lambda i: (0, 0)),
            pl.BlockSpec((1, h), lambda i: (0, 0)),
            pl.BlockSpec((h, e), lambda i: (0, 0)),
            pl.BlockSpec((1, e), lambda i: (0, 0)),
        ],
        out_specs=[
            pl.BlockSpec((BT, TOP_K), lambda i: (i, 0)),
            pl.BlockSpec((BT, TOP_K), lambda i: (i, 0)),
        ],
        out_shape=[
            jax.ShapeDtypeStruct((b, TOP_K), jnp.int32),
            jax.ShapeDtypeStruct((b, TOP_K), jnp.float32),
        ],
    )(x, w1, b1, w2, b2, w3, b3)


def kernel(prompt_embedding, W1, b1, W2, b2, W3, b3):
    idx, wgt = _run(prompt_embedding.astype(jnp.float32), W1,
                    b1.reshape(1, -1), W2, b2.reshape(1, -1), W3,
                    b3.reshape(1, -1))
    return idx, wgt, idx[:, 0]
